# baseline (speedup 1.0000x reference)
import functools

import jax
import jax.numpy as jnp
from jax import lax
from jax.experimental import pallas as pl
from jax.experimental.pallas import tpu as pltpu
from jax.experimental.pallas import tpu_sc as plsc

NU, NM, H, E, L = 50000, 10000, 128, 320000, 100000

NCORE, NSUB, LANES = 2, 16, 16
E_PAD = 323584
PAD_DST = 1 << 28


def _make_seg(width, n_dst, range_size, passes, gather, partial, BLK=128):
    assert range_size % 8 == 0
    NBLK_TOT = E_PAD // BLK
    nch = -(-range_size // BLK)
    rtail = range_size - (nch - 1) * BLK
    nch_full = nch if rtail == BLK else nch - 1
    gtail = n_dst % BLK
    nblks = NBLK_TOT // ((NCORE if partial else 1) * NSUB)
    esl = nblks * BLK
    mesh = plsc.VectorSubcoreMesh(core_axis_name="c", subcore_axis_name="s")

    if partial:
        out_type = [jax.ShapeDtypeStruct((n_dst, width), jnp.float32)
                    for _ in range(NCORE)]
    else:
        out_type = jax.ShapeDtypeStruct((n_dst, width), jnp.float32)

    scratch = [
        pltpu.VMEM((BLK,), jnp.int32),
        pltpu.VMEM((BLK,), jnp.int32),
        pltpu.VMEM((BLK,), jnp.int32),
        pltpu.VMEM((BLK,), jnp.int32),
        pltpu.VMEM((BLK,), jnp.int32),
        pltpu.VMEM((BLK,), jnp.int32),
        pltpu.VMEM((BLK,), jnp.int32),
        pltpu.VMEM((BLK, width), jnp.float32),
        pltpu.VMEM((BLK, width), jnp.float32),
        pltpu.VMEM((BLK, width), jnp.float32),
        pltpu.SemaphoreType.DMA,
        pltpu.SemaphoreType.DMA,
        pltpu.SemaphoreType.DMA,
        pltpu.SemaphoreType.DMA,
        pltpu.SemaphoreType.DMA,
        pltpu.SemaphoreType.DMA,
        pltpu.SemaphoreType.DMA,
        pltpu.VMEM_SHARED((range_size + 64, width), jnp.float32),
    ]

    def body(src_hbm, dst_hbm, table_hbm, *rest):
        nout = NCORE if partial else 1
        outs = rest[:nout]
        rest = rest[nout:]
        bsrc = rest[0:3]
        bdst = rest[3:7]
        rows = rest[7:10]
        gsem = rest[10:13]
        isem = rest[13:15]
        ssem = rest[15:17]
        shared = rest[17]
        cid = lax.axis_index("c")
        sid = lax.axis_index("s")
        ebase = ((cid * NSUB + sid) if partial else sid) * esl

        zf16 = jnp.zeros((LANES,), jnp.float32)

        def fill(buf, val, nrows):
            def fz(i, _):
                for k in range(width // LANES):
                    buf[i, pl.ds(k * LANES, LANES)] = zf16 + val
                return 0
            lax.fori_loop(0, nrows, fz, 0)

        if not gather:
            fill(rows[1], 1.0, BLK)

        def issue_idx(b, t):
            off = ebase + b * BLK
            if gather:
                pltpu.async_copy(src_hbm.at[pl.ds(off, BLK)], bsrc[t % 3],
                                 isem[t % 2])
            pltpu.async_copy(dst_hbm.at[pl.ds(off, BLK)], bdst[t % 4],
                             isem[t % 2])

        def wait_idx(t):
            if gather:
                pltpu.make_async_copy(src_hbm.at[pl.ds(0, BLK)],
                                      bsrc[t % 3], isem[t % 2]).wait()
            pltpu.make_async_copy(dst_hbm.at[pl.ds(0, BLK)], bdst[t % 4],
                                  isem[t % 2]).wait()

        def issue_gather(t):
            pltpu.async_copy(table_hbm.at[bsrc[t % 3]], rows[t % 3],
                             gsem[t % 3])

        def wait_gather(t):
            pltpu.make_async_copy(table_hbm.at[bsrc[t % 3]], rows[t % 3],
                                  gsem[t % 3]).wait()

        for p in range(passes):
            lo = 0 if partial else (cid * passes + p) * range_size
            lov = jnp.zeros((LANES,), jnp.int32) + lo
            rngv = jnp.zeros((LANES,), jnp.int32) + range_size
            m63 = jnp.zeros((LANES,), jnp.int32) + 63

            def transform(t):
                for k in range(BLK // LANES):
                    d = bdst[t % 4][pl.ds(k * LANES, LANES)]
                    m = (d >= lov) & (d < lov + rngv)
                    bdst[t % 4][pl.ds(k * LANES, LANES)] = jnp.where(
                        m, d - lov, rngv + (d & m63))

            def issue_scatter(t):
                grows = rows[t % 3] if gather else rows[1]
                pltpu.async_copy(grows, shared.at[bdst[t % 4]],
                                 ssem[t % 2], add=True)

            def wait_scatter(t):
                grows = rows[t % 3] if gather else rows[1]
                pltpu.make_async_copy(grows, shared.at[bdst[t % 4]],
                                     ssem[t % 2]).wait()

            fill(rows[0], 0.0, BLK)
            for j in range(-(-nch // NSUB)):
                c = sid + j * NSUB

                @pl.when(c < nch_full)
                def _():
                    pltpu.sync_copy(rows[0], shared.at[pl.ds(c * BLK, BLK)])
                if rtail != BLK:
                    @pl.when(c == nch - 1)
                    def _():
                        pltpu.sync_copy(rows[0].at[pl.ds(0, rtail)],
                                        shared.at[pl.ds(c * BLK, rtail)])
            plsc.subcore_barrier()

            issue_idx(0, 0)
            issue_idx(1, 1)
            if gather:
                wait_idx(0)
                issue_gather(0)
                issue_idx(2, 2)
                wait_idx(1)
                issue_gather(1)

            def mac(mj, _):
                for t in range(12):
                    j = mj * 12 + t
                    if gather:
                        @pl.when(j < nblks)
                        def _(t=t):
                            wait_gather(t)
                            transform(t)
                            issue_scatter(t)

                        @pl.when((j >= 1) & (j - 1 < nblks))
                        def _(t=t):
                            wait_scatter(t - 1)

                        @pl.when(j + 2 < nblks)
                        def _(t=t):
                            wait_idx(t + 2)
                            issue_gather(t + 2)

                        @pl.when(j + 3 < nblks)
                        def _(t=t):
                            issue_idx(j + 3, t + 3)
                    else:
                        @pl.when(j < nblks)
                        def _(t=t):
                            wait_idx(t)
                            transform(t)
                            issue_scatter(t)

                        @pl.when((j >= 1) & (j - 1 < nblks))
                        def _(t=t):
                            wait_scatter(t - 1)

                        @pl.when(j + 2 < nblks)
                        def _(t=t):
                            issue_idx(j + 2, t + 2)
                return 0
            lax.fori_loop(0, -(-nblks // 12), mac, 0)
            if nblks % 12 == 0:
                wait_scatter(nblks - 1)
            plsc.subcore_barrier()

            for j in range(-(-nch // NSUB)):
                c = sid + j * NSUB
                start = lo + c * BLK
                for ci in range(len(outs)):
                    here = (cid == ci) if partial else (c >= 0)

                    @pl.when(here & (c < nch_full)
                             & (start + BLK <= n_dst))
                    def _(ci=ci):
                        pltpu.sync_copy(shared.at[pl.ds(c * BLK, BLK)],
                                        outs[ci].at[pl.ds(start, BLK)])
                    if rtail != BLK:
                        @pl.when(here & (c == nch - 1)
                                 & (start + rtail <= n_dst))
                        def _(ci=ci):
                            pltpu.sync_copy(
                                shared.at[pl.ds(c * BLK, rtail)],
                                outs[ci].at[pl.ds(start, rtail)])
                    if gtail:
                        @pl.when(here & (c < nch_full) & (start < n_dst)
                                 & (start + BLK > n_dst))
                        def _(ci=ci):
                            pltpu.sync_copy(
                                shared.at[pl.ds(c * BLK, gtail)],
                                outs[ci].at[pl.ds(start, gtail)])
            if p + 1 < passes:
                plsc.subcore_barrier()

    return functools.partial(pl.kernel, mesh=mesh, out_type=out_type,
                             scratch_types=scratch)(body)


_seg_u = _make_seg(H, NU, 12544, 2, gather=True, partial=False, BLK=64)
_seg_m = _make_seg(H, NM, NM, 1, gather=True, partial=True)


def _mm_body(x_ref, w_ref, o_ref):
    o_ref[...] = jnp.dot(x_ref[...], w_ref[...],
                         preferred_element_type=jnp.float32)


def _matmul(x, w, block=1000):
    n, k = x.shape
    h = w.shape[1]
    return pl.pallas_call(
        _mm_body,
        grid=(n // block,),
        in_specs=[pl.BlockSpec((block, k), lambda i: (i, 0)),
                  pl.BlockSpec((k, h), lambda i: (0, 0))],
        out_specs=pl.BlockSpec((block, h), lambda i: (i, 0)),
        out_shape=jax.ShapeDtypeStruct((n, h), jnp.float32),
    )(x, w)


def _combine_body(relu, two, a_ref, *rest):
    if two:
        a2_ref, ic_ref, x_ref, w_ref, b_ref, o_ref = rest
        asum = a_ref[...] + a2_ref[...]
    else:
        ic_ref, x_ref, w_ref, b_ref, o_ref = rest
        asum = a_ref[...]
    acc = asum * ic_ref[...] + jnp.dot(
        x_ref[...], w_ref[...], preferred_element_type=jnp.float32) + b_ref[...]
    o_ref[...] = jnp.maximum(acc, 0.0) if relu else acc


def _combine(asums, inv_cnt, x, w, b, relu, block=1000):
    n, k = x.shape
    h = w.shape[1]
    two = len(asums) == 2
    aspecs = [pl.BlockSpec((block, h), lambda i: (i, 0)) for _ in asums]
    return pl.pallas_call(
        functools.partial(_combine_body, relu, two),
        grid=(n // block,),
        in_specs=aspecs + [
            pl.BlockSpec((block, 1), lambda i: (i, 0)),
            pl.BlockSpec((block, k), lambda i: (i, 0)),
            pl.BlockSpec((k, h), lambda i: (0, 0)),
            pl.BlockSpec((1, h), lambda i: (0, 0))],
        out_specs=pl.BlockSpec((block, h), lambda i: (i, 0)),
        out_shape=jax.ShapeDtypeStruct((n, h), jnp.float32),
    )(*asums, inv_cnt, x, w, b.reshape(1, h))


def kernel(user_id, movie_id, x_movie, rates_src, rates_dst, label_user,
           label_movie, user_emb, movie_emb,
           Wl1_mu, Wr1_mu, b1_mu, Wl1_um, Wr1_um, b1_um,
           Wl2_mu, Wr2_mu, b2_mu, Wl2_um, Wr2_um, b2_um,
           Wl3_mu, Wr3_mu, b3_mu, Wl3_um, Wr3_um, b3_um,
           Wh_u, bh_u, Wh_m, bh_m):
    xu = user_emb
    xm = jnp.concatenate([movie_emb, x_movie], axis=-1)

    npad = E_PAD - E
    rs = rates_src.astype(jnp.int32)
    rd = rates_dst.astype(jnp.int32)
    pad0 = jnp.zeros((npad,), jnp.int32)
    padT = jnp.full((npad,), PAD_DST, jnp.int32)
    rs0 = jnp.concatenate([rs, pad0])
    rsT = jnp.concatenate([rs, padT])
    rd0 = jnp.concatenate([rd, pad0])
    rdT = jnp.concatenate([rd, padT])

    ones = jnp.ones((E,), jnp.float32)
    cnt_u = jax.ops.segment_sum(ones, rs, num_segments=NU)
    cnt_m = jax.ops.segment_sum(ones, rd, num_segments=NM)
    icu = (1.0 / jnp.maximum(cnt_u, 1.0)).reshape(NU, 1)
    icm = (1.0 / jnp.maximum(cnt_m, 1.0)).reshape(NM, 1)

    def layer(xu_in, xm_in, Wl_mu, Wr_mu, b_mu, Wl_um, Wr_um, b_um, relu):
        au = _seg_u(rd0, rsT, _matmul(xm_in, Wl_mu))
        am = _seg_m(rs0, rdT, _matmul(xu_in, Wl_um))
        u = _combine([au], icu, xu_in, Wr_mu, b_mu, relu=relu)
        m = _combine(list(am), icm, xm_in, Wr_um, b_um, relu=relu)
        return u, m

    u1, m1 = layer(xu, xm, Wl1_mu, Wr1_mu, b1_mu, Wl1_um, Wr1_um, b1_um, True)
    u2, m2 = layer(u1, m1, Wl2_mu, Wr2_mu, b2_mu, Wl2_um, Wr2_um, b2_um, True)
    u3, m3 = layer(u2, m2, Wl3_mu, Wr3_mu, b3_mu, Wl3_um, Wr3_um, b3_um,
                   False)

    zu = _combine([jnp.zeros((NU, H), jnp.float32)], icu, u3, Wh_u, bh_u,
                  relu=False)
    zm = _combine([jnp.zeros((NM, H), jnp.float32)], icm, m3, Wh_m, bh_m,
                  relu=False)

    return (zu[label_user] * zm[label_movie]).sum(axis=1)


# speedup vs baseline: 1.0435x; 1.0435x over previous
import functools

import jax
import jax.numpy as jnp
from jax import lax
from jax.experimental import pallas as pl
from jax.experimental.pallas import tpu as pltpu
from jax.experimental.pallas import tpu_sc as plsc

NU, NM, H, E, L = 50000, 10000, 128, 320000, 100000
_USE_SC_DECODER = True

NCORE, NSUB, LANES = 2, 16, 16
E_PAD = 323584
PAD_DST = 1 << 28


def _make_seg(width, n_dst, range_size, passes, gather, partial, BLK=128):
    assert range_size % 8 == 0
    NBLK_TOT = E_PAD // BLK
    nch = -(-range_size // BLK)
    rtail = range_size - (nch - 1) * BLK
    nch_full = nch if rtail == BLK else nch - 1
    gtail = n_dst % BLK
    nblks = NBLK_TOT // ((NCORE if partial else 1) * NSUB)
    esl = nblks * BLK
    mesh = plsc.VectorSubcoreMesh(core_axis_name="c", subcore_axis_name="s")

    if partial:
        out_type = [jax.ShapeDtypeStruct((n_dst, width), jnp.float32)
                    for _ in range(NCORE)]
    else:
        out_type = jax.ShapeDtypeStruct((n_dst, width), jnp.float32)

    scratch = [
        pltpu.VMEM((BLK,), jnp.int32),
        pltpu.VMEM((BLK,), jnp.int32),
        pltpu.VMEM((BLK,), jnp.int32),
        pltpu.VMEM((BLK,), jnp.int32),
        pltpu.VMEM((BLK,), jnp.int32),
        pltpu.VMEM((BLK,), jnp.int32),
        pltpu.VMEM((BLK,), jnp.int32),
        pltpu.VMEM((BLK, width), jnp.float32),
        pltpu.VMEM((BLK, width), jnp.float32),
        pltpu.VMEM((BLK, width), jnp.float32),
        pltpu.SemaphoreType.DMA,
        pltpu.SemaphoreType.DMA,
        pltpu.SemaphoreType.DMA,
        pltpu.SemaphoreType.DMA,
        pltpu.SemaphoreType.DMA,
        pltpu.SemaphoreType.DMA,
        pltpu.SemaphoreType.DMA,
        pltpu.VMEM_SHARED((range_size + 64, width), jnp.float32),
    ]

    def body(*args):
        if gather:
            src_hbm, dst_hbm, table_hbm = args[:3]
            rest = args[3:]
        else:
            src_hbm = table_hbm = None
            dst_hbm = args[0]
            rest = args[1:]
        nout = NCORE if partial else 1
        outs = rest[:nout]
        rest = rest[nout:]
        bsrc = rest[0:3]
        bdst = rest[3:7]
        rows = rest[7:10]
        gsem = rest[10:13]
        isem = rest[13:15]
        ssem = rest[15:17]
        shared = rest[17]
        cid = lax.axis_index("c")
        sid = lax.axis_index("s")
        ebase = ((cid * NSUB + sid) if partial else sid) * esl

        zf16 = jnp.zeros((LANES,), jnp.float32)

        def fill(buf, val, nrows):
            def fz(i, _):
                for k in range(width // LANES):
                    buf[i, pl.ds(k * LANES, LANES)] = zf16 + val
                return 0
            lax.fori_loop(0, nrows, fz, 0)

        if not gather:
            fill(rows[1], 1.0, BLK)

        def issue_idx(b, t):
            off = ebase + b * BLK
            if gather:
                pltpu.async_copy(src_hbm.at[pl.ds(off, BLK)], bsrc[t % 3],
                                 isem[t % 2])
            pltpu.async_copy(dst_hbm.at[pl.ds(off, BLK)], bdst[t % 4],
                             isem[t % 2])

        def wait_idx(t):
            if gather:
                pltpu.make_async_copy(src_hbm.at[pl.ds(0, BLK)],
                                      bsrc[t % 3], isem[t % 2]).wait()
            pltpu.make_async_copy(dst_hbm.at[pl.ds(0, BLK)], bdst[t % 4],
                                  isem[t % 2]).wait()

        def issue_gather(t):
            pltpu.async_copy(table_hbm.at[bsrc[t % 3]], rows[t % 3],
                             gsem[t % 3])

        def wait_gather(t):
            pltpu.make_async_copy(table_hbm.at[bsrc[t % 3]], rows[t % 3],
                                  gsem[t % 3]).wait()

        for p in range(passes):
            lo = 0 if partial else (cid * passes + p) * range_size
            lov = jnp.zeros((LANES,), jnp.int32) + lo
            rngv = jnp.zeros((LANES,), jnp.int32) + range_size
            m63 = jnp.zeros((LANES,), jnp.int32) + 63

            def transform(t):
                for k in range(BLK // LANES):
                    d = bdst[t % 4][pl.ds(k * LANES, LANES)]
                    m = (d >= lov) & (d < lov + rngv)
                    bdst[t % 4][pl.ds(k * LANES, LANES)] = jnp.where(
                        m, d - lov, rngv + (d & m63))

            def issue_scatter(t):
                grows = rows[t % 3] if gather else rows[1]
                pltpu.async_copy(grows, shared.at[bdst[t % 4]],
                                 ssem[t % 2], add=True)

            def wait_scatter(t):
                grows = rows[t % 3] if gather else rows[1]
                pltpu.make_async_copy(grows, shared.at[bdst[t % 4]],
                                     ssem[t % 2]).wait()

            fill(rows[0], 0.0, BLK)
            for j in range(-(-nch // NSUB)):
                c = sid + j * NSUB

                @pl.when(c < nch_full)
                def _():
                    pltpu.sync_copy(rows[0], shared.at[pl.ds(c * BLK, BLK)])
                if rtail != BLK:
                    @pl.when(c == nch - 1)
                    def _():
                        pltpu.sync_copy(rows[0].at[pl.ds(0, rtail)],
                                        shared.at[pl.ds(c * BLK, rtail)])
            plsc.subcore_barrier()

            issue_idx(0, 0)
            issue_idx(1, 1)
            if gather:
                wait_idx(0)
                issue_gather(0)
                issue_idx(2, 2)
                wait_idx(1)
                issue_gather(1)

            def mac(mj, _):
                for t in range(12):
                    j = mj * 12 + t
                    if gather:
                        @pl.when(j < nblks)
                        def _(t=t):
                            wait_gather(t)
                            transform(t)
                            issue_scatter(t)

                        @pl.when((j >= 1) & (j - 1 < nblks))
                        def _(t=t):
                            wait_scatter(t - 1)

                        @pl.when(j + 2 < nblks)
                        def _(t=t):
                            wait_idx(t + 2)
                            issue_gather(t + 2)

                        @pl.when(j + 3 < nblks)
                        def _(t=t):
                            issue_idx(j + 3, t + 3)
                    else:
                        @pl.when(j < nblks)
                        def _(t=t):
                            wait_idx(t)
                            transform(t)
                            issue_scatter(t)

                        @pl.when((j >= 1) & (j - 1 < nblks))
                        def _(t=t):
                            wait_scatter(t - 1)

                        @pl.when(j + 2 < nblks)
                        def _(t=t):
                            issue_idx(j + 2, t + 2)
                return 0
            lax.fori_loop(0, -(-nblks // 12), mac, 0)
            if nblks % 12 == 0:
                wait_scatter(nblks - 1)
            plsc.subcore_barrier()

            for j in range(-(-nch // NSUB)):
                c = sid + j * NSUB
                start = lo + c * BLK
                for ci in range(len(outs)):
                    here = (cid == ci) if partial else (c >= 0)

                    @pl.when(here & (c < nch_full)
                             & (start + BLK <= n_dst))
                    def _(ci=ci):
                        pltpu.sync_copy(shared.at[pl.ds(c * BLK, BLK)],
                                        outs[ci].at[pl.ds(start, BLK)])
                    if rtail != BLK:
                        @pl.when(here & (c == nch - 1)
                                 & (start + rtail <= n_dst))
                        def _(ci=ci):
                            pltpu.sync_copy(
                                shared.at[pl.ds(c * BLK, rtail)],
                                outs[ci].at[pl.ds(start, rtail)])
                    if gtail:
                        @pl.when(here & (c < nch_full) & (start < n_dst)
                                 & (start + BLK > n_dst))
                        def _(ci=ci):
                            pltpu.sync_copy(
                                shared.at[pl.ds(c * BLK, gtail)],
                                outs[ci].at[pl.ds(start, gtail)])
            if p + 1 < passes:
                plsc.subcore_barrier()

    return functools.partial(pl.kernel, mesh=mesh, out_type=out_type,
                             scratch_types=scratch)(body)


_seg_u = _make_seg(H, NU, 12544, 2, gather=True, partial=False, BLK=64)
_seg_m = _make_seg(H, NM, NM, 1, gather=True, partial=True)
_cnt_u = _make_seg(16, NU, 25088, 1, gather=False, partial=False)
_cnt_m = _make_seg(16, NM, NM, 1, gather=False, partial=True)

L_PAD = 100352
_DCH = 1024
_DSB = 128


def _make_decoder():
    nchk = L_PAD // _DCH
    mesh = plsc.VectorSubcoreMesh(core_axis_name="c", subcore_axis_name="s")

    scratch = [
        pltpu.VMEM((_DCH,), jnp.int32),
        pltpu.VMEM((_DCH,), jnp.int32),
        pltpu.VMEM((_DSB, H), jnp.float32),
        pltpu.VMEM((_DSB, H), jnp.float32),
        pltpu.VMEM((_DSB, H), jnp.float32),
        pltpu.VMEM((_DSB, H), jnp.float32),
        pltpu.VMEM((_DCH,), jnp.float32),
        pltpu.SemaphoreType.DMA,
        pltpu.SemaphoreType.DMA,
    ]

    @functools.partial(
        pl.kernel, mesh=mesh,
        out_type=jax.ShapeDtypeStruct((L_PAD,), jnp.float32),
        scratch_types=scratch,
    )
    def decoder(lu_hbm, lm_hbm, zu_hbm, zm_hbm, out_hbm,
                lu_st, lm_st, ru0, ru1, rm0, rm1, out_st, sem0, sem1):
        cid = lax.axis_index("c")
        sid = lax.axis_index("s")
        w = cid * NSUB + sid
        ru = (ru0, ru1)
        rm = (rm0, rm1)
        sem = (sem0, sem1)
        iota16 = lax.iota(jnp.int32, LANES)
        gdn = lax.GatherDimensionNumbers(
            offset_dims=(), collapsed_slice_dims=(0,), start_index_map=(0,))

        def lane_perm(v, idx):
            return lax.gather(v, idx.reshape(LANES, 1), gdn, (1,),
                              mode=lax.GatherScatterMode.PROMISE_IN_BOUNDS)

        def issue(sb, par):
            pltpu.async_copy(zu_hbm.at[lu_st.at[pl.ds(sb * _DSB, _DSB)]],
                             ru[par], sem[par])
            pltpu.async_copy(zm_hbm.at[lm_st.at[pl.ds(sb * _DSB, _DSB)]],
                             rm[par], sem[par])

        def wait(par):
            pltpu.make_async_copy(zu_hbm.at[lu_st.at[pl.ds(0, _DSB)]],
                                  ru[par], sem[par]).wait()
            pltpu.make_async_copy(zm_hbm.at[lm_st.at[pl.ds(0, _DSB)]],
                                  rm[par], sem[par]).wait()

        def chunk(ch):
            base = ch * _DCH
            pltpu.sync_copy(lu_hbm.at[pl.ds(base, _DCH)], lu_st)
            pltpu.sync_copy(lm_hbm.at[pl.ds(base, _DCH)], lm_st)
            issue(0, 0)

            def sub(sb, _):
                for par in range(2):
                    @pl.when(sb % 2 == par)
                    def _(par=par):
                        @pl.when(sb + 1 < _DCH // _DSB)
                        def _(par=par):
                            issue(sb + 1, 1 - par)
                        wait(par)

                        def grp(g, _):
                            res = jnp.zeros((LANES,), jnp.float32)
                            for q in range(LANES):
                                r = g * LANES + q
                                acc = (ru[par][r, pl.ds(0, LANES)]
                                       * rm[par][r, pl.ds(0, LANES)])
                                for k in range(1, H // LANES):
                                    acc = acc + (
                                        ru[par][r, pl.ds(k * LANES, LANES)]
                                        * rm[par][r, pl.ds(k * LANES, LANES)])
                                for sh in (1, 2, 4, 8):
                                    g2 = lane_perm(
                                        acc, jnp.maximum(iota16 - sh, 0))
                                    acc = acc + jnp.where(
                                        iota16 >= sh, g2,
                                        jnp.zeros((LANES,), jnp.float32))
                                tot = lane_perm(acc, iota16 * 0 + 15)
                                res = jnp.where(iota16 == q, tot, res)
                            out_st[pl.ds(sb * _DSB + g * LANES, LANES)] = res
                            return 0
                        lax.fori_loop(0, _DSB // LANES, grp, 0)
                return 0
            lax.fori_loop(0, _DCH // _DSB, sub, 0)
            pltpu.sync_copy(out_st, out_hbm.at[pl.ds(base, _DCH)])

        def tloop(t, _):
            ch = w + t * NCORE * NSUB

            @pl.when(ch < nchk)
            def _():
                chunk(ch)
            return 0
        lax.fori_loop(0, -(-nchk // (NCORE * NSUB)), tloop, 0)

    return decoder


_decoder = _make_decoder()


def _mm_body(x_ref, w_ref, o_ref):
    o_ref[...] = jnp.dot(x_ref[...], w_ref[...],
                         preferred_element_type=jnp.float32)


def _matmul(x, w, block=1000):
    n, k = x.shape
    h = w.shape[1]
    return pl.pallas_call(
        _mm_body,
        grid=(n // block,),
        in_specs=[pl.BlockSpec((block, k), lambda i: (i, 0)),
                  pl.BlockSpec((k, h), lambda i: (0, 0))],
        out_specs=pl.BlockSpec((block, h), lambda i: (i, 0)),
        out_shape=jax.ShapeDtypeStruct((n, h), jnp.float32),
    )(x, w)


def _combine_body(relu, na, nc, *refs):
    a_refs = refs[:na]
    c_refs = refs[na:na + nc]
    x_ref, w_ref, b_ref, o_ref = refs[na + nc:]
    asum = a_refs[0][...]
    for a in a_refs[1:]:
        asum = asum + a[...]
    cnt = c_refs[0][:, 0:1]
    for c in c_refs[1:]:
        cnt = cnt + c[:, 0:1]
    ic = 1.0 / jnp.maximum(cnt, 1.0)
    acc = asum * ic + jnp.dot(
        x_ref[...], w_ref[...], preferred_element_type=jnp.float32) + b_ref[...]
    o_ref[...] = jnp.maximum(acc, 0.0) if relu else acc


def _combine(asums, cnts, x, w, b, relu, block=1000):
    n, k = x.shape
    h = w.shape[1]
    specs = ([pl.BlockSpec((block, h), lambda i: (i, 0)) for _ in asums]
             + [pl.BlockSpec((block, 1), lambda i: (i, 0)) for _ in cnts]
             + [pl.BlockSpec((block, k), lambda i: (i, 0)),
                pl.BlockSpec((k, h), lambda i: (0, 0)),
                pl.BlockSpec((1, h), lambda i: (0, 0))])
    return pl.pallas_call(
        functools.partial(_combine_body, relu, len(asums), len(cnts)),
        grid=(n // block,),
        in_specs=specs,
        out_specs=pl.BlockSpec((block, h), lambda i: (i, 0)),
        out_shape=jax.ShapeDtypeStruct((n, h), jnp.float32),
    )(*asums, *cnts, x, w, b.reshape(1, h))


def kernel(user_id, movie_id, x_movie, rates_src, rates_dst, label_user,
           label_movie, user_emb, movie_emb,
           Wl1_mu, Wr1_mu, b1_mu, Wl1_um, Wr1_um, b1_um,
           Wl2_mu, Wr2_mu, b2_mu, Wl2_um, Wr2_um, b2_um,
           Wl3_mu, Wr3_mu, b3_mu, Wl3_um, Wr3_um, b3_um,
           Wh_u, bh_u, Wh_m, bh_m):
    xu = user_emb
    xm = jnp.concatenate([movie_emb, x_movie], axis=-1)

    npad = E_PAD - E
    rs = rates_src.astype(jnp.int32)
    rd = rates_dst.astype(jnp.int32)
    pad0 = jnp.zeros((npad,), jnp.int32)
    padT = jnp.full((npad,), PAD_DST, jnp.int32)
    rs0 = jnp.concatenate([rs, pad0])
    rsT = jnp.concatenate([rs, padT])
    rd0 = jnp.concatenate([rd, pad0])
    rdT = jnp.concatenate([rd, padT])

    ones = jnp.ones((E,), jnp.float32)
    cu = [jax.ops.segment_sum(ones, rs, num_segments=NU).reshape(NU, 1)]
    cm = [jax.ops.segment_sum(ones, rd, num_segments=NM).reshape(NM, 1)]

    def layer(xu_in, xm_in, Wl_mu, Wr_mu, b_mu, Wl_um, Wr_um, b_um, relu):
        au = _seg_u(rd0, rsT, _matmul(xm_in, Wl_mu))
        am = _seg_m(rs0, rdT, _matmul(xu_in, Wl_um))
        u = _combine([au], cu, xu_in, Wr_mu, b_mu, relu=relu)
        m = _combine(list(am), cm, xm_in, Wr_um, b_um, relu=relu)
        return u, m

    u1, m1 = layer(xu, xm, Wl1_mu, Wr1_mu, b1_mu, Wl1_um, Wr1_um, b1_um, True)
    u2, m2 = layer(u1, m1, Wl2_mu, Wr2_mu, b2_mu, Wl2_um, Wr2_um, b2_um, True)
    u3, m3 = layer(u2, m2, Wl3_mu, Wr3_mu, b3_mu, Wl3_um, Wr3_um, b3_um,
                   False)

    zu = _combine([jnp.zeros((NU, H), jnp.float32)], cu, u3, Wh_u, bh_u,
                  relu=False)
    zm = _combine([jnp.zeros((NM, H), jnp.float32)],
                  [jnp.zeros((NM, 1), jnp.float32)], m3, Wh_m, bh_m,
                  relu=False)

    if _USE_SC_DECODER:
        lpad = jnp.zeros((L_PAD - L,), jnp.int32)
        lu_p = jnp.concatenate([label_user.astype(jnp.int32), lpad])
        lm_p = jnp.concatenate([label_movie.astype(jnp.int32), lpad])
        return _decoder(lu_p, lm_p, zu, zm)[:L]
    return (zu[label_user] * zm[label_movie]).sum(axis=1)
